# Initial kernel scaffold; baseline (speedup 1.0000x reference)
import jax
import jax.numpy as jnp
from jax.experimental import pallas as pl


def kernel(u, W_qkey, W_wkey, W_value, W_out, W_gate, b_gate, W_vsa, R_bits, mem_keys, mem_vals, mem_tags):
    raise NotImplementedError("write your pallas kernel here")


# speedup vs baseline: 1.3971x; 1.3971x over previous
import functools

import jax
import jax.numpy as jnp
import numpy as np
from jax.experimental import pallas as pl
from jax.experimental.pallas import tpu as pltpu

B, S, D = 2, 4096, 1024
H, NBUCKETS, A = 2, 65536, 2
KD, MD, VD = 32, 64, 32
NBITS = 16
ETA = 0.1
THR = 0.5
VSA_W = 1.0

N = B * S
BLK = 512
NBLK = N // BLK
RSQ = 1.0 / np.sqrt(KD)
NROWS = H * NBUCKETS * A


def _proj_body(x_ref, wcat_ref, rcat_ref, wvsa_ref, bg_ref, f_ref, i_ref):
    x = x_ref[...]
    P = jax.lax.dot_general(x, wcat_ref[...], (((1,), (0,)), ((), ())),
                            preferred_element_type=jnp.float32)
    tag = P[:, 0:32]
    wkey = P[:, 32:64]
    val = P[:, 64:128]
    glog = P[:, 128:129] + bg_ref[0, 0]
    g = jax.nn.sigmoid(glog)
    gE = ETA * jnp.where(g > THR, g, 0.0)
    pb = jax.lax.dot_general(tag, rcat_ref[...], (((1,), (0,)), ((), ())),
                             preferred_element_type=jnp.float32)[:, 0:32]
    col = jax.lax.broadcasted_iota(jnp.int32, (BLK, 32), 1)
    powers = jnp.left_shift(jnp.int32(1), jnp.bitwise_and(col, 15))
    pbits = jnp.where(pb > 0, powers, 0)
    idx0 = jnp.sum(pbits[:, 0:16], axis=1, keepdims=True)
    idx1 = jnp.sum(pbits[:, 16:32], axis=1, keepdims=True)
    wv = wvsa_ref[...][:, 0:32]
    wtag = jnp.tanh(jax.lax.dot_general(wkey, wv, (((1,), (0,)), ((), ())),
                                        preferred_element_type=jnp.float32))
    qtag = jnp.tanh(jax.lax.dot_general(tag, wv, (((1,), (0,)), ((), ())),
                                        preferred_element_type=jnp.float32))
    f_ref[...] = jnp.concatenate(
        [tag, wkey, wtag, qtag, val, gE, jnp.zeros((BLK, 63), jnp.float32)], axis=1)
    zi = jnp.zeros((BLK, 1), jnp.int32)
    i_ref[...] = jnp.concatenate([idx0, idx1, zi, zi, zi, zi, zi, zi], axis=1)


def _proj(x, Wcat, Rcat, Wvsa, bg):
    return pl.pallas_call(
        _proj_body,
        grid=(NBLK,),
        in_specs=[
            pl.BlockSpec((BLK, D), lambda i: (i, 0)),
            pl.BlockSpec((D, 256), lambda i: (0, 0)),
            pl.BlockSpec((32, 128), lambda i: (0, 0)),
            pl.BlockSpec((32, 128), lambda i: (0, 0)),
            pl.BlockSpec((1, 1), lambda i: (0, 0), memory_space=pltpu.SMEM),
        ],
        out_specs=[
            pl.BlockSpec((BLK, 256), lambda i: (i, 0)),
            pl.BlockSpec((BLK, 8), lambda i: (i, 0)),
        ],
        out_shape=[
            jax.ShapeDtypeStruct((N, 256), jnp.float32),
            jax.ShapeDtypeStruct((N, 8), jnp.int32),
        ],
    )(x, Wcat, Rcat, Wvsa, bg)


def _delta_body(f_ref, i_ref, k0_ref, v0_ref, t0_ref, d_ref, w_ref, s_ref):
    F = f_ref[...]
    tag = F[:, 0:32]
    wkey = F[:, 32:64]
    wtag = F[:, 64:96]
    qtag = F[:, 96:128]
    val = F[:, 128:192]
    gE = F[:, 192:193]
    K0 = k0_ref[...]
    V0 = v0_ref[...]
    T0 = t0_ref[...]
    m00 = jnp.sum(K0[:, 0:32] * wkey, axis=1, keepdims=True)
    m01 = jnp.sum(K0[:, 32:64] * wkey, axis=1, keepdims=True)
    m10 = jnp.sum(K0[:, 64:96] * wkey, axis=1, keepdims=True)
    m11 = jnp.sum(K0[:, 96:128] * wkey, axis=1, keepdims=True)
    slot0 = m01 > m00
    slot1 = m11 > m10
    oldk0 = jnp.where(slot0, K0[:, 32:64], K0[:, 0:32])
    oldk1 = jnp.where(slot1, K0[:, 96:128], K0[:, 64:96])
    oldv0 = jnp.where(slot0, V0[:, 64:128], V0[:, 0:64])
    oldv1 = jnp.where(slot1, V0[:, 192:256], V0[:, 128:192])
    oldt0 = jnp.where(slot0, T0[:, 32:64], T0[:, 0:32])
    oldt1 = jnp.where(slot1, T0[:, 96:128], T0[:, 64:96])
    base = jnp.concatenate([wkey, val, wtag], axis=1)
    d0 = gE * (base - jnp.concatenate([oldk0, oldv0, oldt0], axis=1))
    d1 = gE * (base - jnp.concatenate([oldk1, oldv1, oldt1], axis=1))
    d_ref[...] = jnp.concatenate([d0, d1], axis=1)
    idx = i_ref[...]
    wrow0 = idx[:, 0:1] * 2 + slot0.astype(jnp.int32)
    wrow1 = 131072 + idx[:, 1:2] * 2 + slot1.astype(jnp.int32)
    zi = jnp.zeros((BLK, 1), jnp.int32)
    w_ref[...] = jnp.concatenate([wrow0, wrow1, zi, zi, zi, zi, zi, zi], axis=1)
    s00 = jnp.sum(K0[:, 0:32] * tag, 1, keepdims=True) * RSQ + \
        VSA_W * jnp.sum(T0[:, 0:32] * qtag, 1, keepdims=True)
    s01 = jnp.sum(K0[:, 32:64] * tag, 1, keepdims=True) * RSQ + \
        VSA_W * jnp.sum(T0[:, 32:64] * qtag, 1, keepdims=True)
    s10 = jnp.sum(K0[:, 64:96] * tag, 1, keepdims=True) * RSQ + \
        VSA_W * jnp.sum(T0[:, 64:96] * qtag, 1, keepdims=True)
    s11 = jnp.sum(K0[:, 96:128] * tag, 1, keepdims=True) * RSQ + \
        VSA_W * jnp.sum(T0[:, 96:128] * qtag, 1, keepdims=True)
    zf = jnp.zeros((BLK, 1), jnp.float32)
    s_ref[...] = jnp.concatenate([s00, s01, s10, s11, zf, zf, zf, zf], axis=1)


def _delta(F, I, K0, V0, T0):
    return pl.pallas_call(
        _delta_body,
        grid=(NBLK,),
        in_specs=[
            pl.BlockSpec((BLK, 256), lambda i: (i, 0)),
            pl.BlockSpec((BLK, 8), lambda i: (i, 0)),
            pl.BlockSpec((BLK, 128), lambda i: (i, 0)),
            pl.BlockSpec((BLK, 256), lambda i: (i, 0)),
            pl.BlockSpec((BLK, 128), lambda i: (i, 0)),
        ],
        out_specs=[
            pl.BlockSpec((BLK, 256), lambda i: (i, 0)),
            pl.BlockSpec((BLK, 8), lambda i: (i, 0)),
            pl.BlockSpec((BLK, 8), lambda i: (i, 0)),
        ],
        out_shape=[
            jax.ShapeDtypeStruct((N, 256), jnp.float32),
            jax.ShapeDtypeStruct((N, 8), jnp.int32),
            jax.ShapeDtypeStruct((N, 8), jnp.float32),
        ],
    )(F, I, K0, V0, T0)


def _combine_body(f_ref, v0_ref, ds_ref, s0_ref, wout_ref, y_ref):
    F = f_ref[...]
    tag = F[:, 0:32]
    qtag = F[:, 96:128]
    V0 = v0_ref[...]
    DS = ds_ref[...]
    s0 = s0_ref[...]
    logits = []
    rvs = []
    for ha in range(4):
        dk = DS[:, ha * 128:ha * 128 + 32]
        dv = DS[:, ha * 128 + 32:ha * 128 + 96]
        dt = DS[:, ha * 128 + 96:ha * 128 + 128]
        lg = s0[:, ha:ha + 1] + jnp.sum(dk * tag, 1, keepdims=True) * RSQ \
            + VSA_W * jnp.sum(dt * qtag, 1, keepdims=True)
        logits.append(lg)
        rvs.append(V0[:, ha * 64:(ha + 1) * 64] + dv)
    lg = jnp.concatenate(logits, axis=1)
    m = jnp.max(lg, axis=1, keepdims=True)
    e = jnp.exp(lg - m)
    attn = e / jnp.sum(e, axis=1, keepdims=True)
    read = (attn[:, 0:1] * rvs[0] + attn[:, 1:2] * rvs[1]
            + attn[:, 2:3] * rvs[2] + attn[:, 3:4] * rvs[3])
    y_ref[...] = jax.lax.dot_general(read, wout_ref[...], (((1,), (0,)), ((), ())),
                                     preferred_element_type=jnp.float32)


def _combine(F, V0, DS, s0, W_out):
    return pl.pallas_call(
        _combine_body,
        grid=(NBLK,),
        in_specs=[
            pl.BlockSpec((BLK, 256), lambda i: (i, 0)),
            pl.BlockSpec((BLK, 256), lambda i: (i, 0)),
            pl.BlockSpec((BLK, 512), lambda i: (i, 0)),
            pl.BlockSpec((BLK, 8), lambda i: (i, 0)),
            pl.BlockSpec((MD, D), lambda i: (0, 0)),
        ],
        out_specs=pl.BlockSpec((BLK, D), lambda i: (i, 0)),
        out_shape=jax.ShapeDtypeStruct((N, D), jnp.float32),
    )(F, V0, DS, s0, W_out)


def kernel(u, W_qkey, W_wkey, W_value, W_out, W_gate, b_gate, W_vsa,
           R_bits, mem_keys, mem_vals, mem_tags):
    x = u.reshape(N, D)
    Wcat = jnp.concatenate([W_qkey, W_wkey, W_value, W_gate], axis=1)
    Wcat = jnp.pad(Wcat, ((0, 0), (0, 256 - 129)))
    Rcat = jnp.pad(jnp.transpose(R_bits, (1, 0, 2)).reshape(KD, H * NBITS),
                   ((0, 0), (0, 128 - H * NBITS)))
    Wvsa = jnp.pad(W_vsa, ((0, 0), (0, 128 - VD)))
    bg = b_gate.reshape(1, 1)

    F, I = _proj(x, Wcat, Rcat, Wvsa, bg)

    idx = I[:, 0:2]
    harr = jnp.arange(H, dtype=jnp.int32)[None, :]
    rbase = harr * (NBUCKETS * A) + idx * A
    rr = (rbase[:, :, None] + jnp.arange(A, dtype=jnp.int32)[None, None, :])
    rrf = rr.reshape(-1)
    mk = mem_keys.reshape(NROWS, KD)
    mv = mem_vals.reshape(NROWS, MD)
    mt = mem_tags.reshape(NROWS, VD)
    K0 = mk[rrf].reshape(N, H * A * KD)
    V0 = mv[rrf].reshape(N, H * A * MD)
    T0 = mt[rrf].reshape(N, H * A * VD)

    dR, WR, s0 = _delta(F, I, K0, V0, T0)

    dflat = dR.reshape(N * H, 128)
    wflat = WR[:, 0:2].reshape(N * H)
    acc = jnp.zeros((NROWS, 128), jnp.float32).at[wflat].add(dflat)
    DS = acc[rrf].reshape(N, H * A * 128)

    y = _combine(F, V0, DS, s0, W_out)
    return y.reshape(B, S, D)
